# baseline (speedup 1.0000x reference)
import functools

import jax
import jax.numpy as jnp
from jax import lax
from jax.experimental import pallas as pl
from jax.experimental.pallas import tpu as pltpu
from jax.experimental.pallas import tpu_sc as plsc

NTOKEN = 100000
DMODEL = 1024
SEQLEN = 2048
BATCH = 4
EPS = 1e-5

NC = 2
NS = 16
NW = NC * NS
NTOK = BATCH * SEQLEN

SC_TOK = NTOK

NSLICE = 4
NTOKK = SC_TOK // NSLICE
HSEQ = SEQLEN // NSLICE
WPB = NW // BATCH
TOK_PER_W = HSEQ // WPB
CH = 16
NCH = TOK_PER_W // CH
NBUF = 6
LOOKAHEAD = 4

BS = 512
SBLK = SEQLEN // BS
GRIDK = NTOKK // BS


def _sc_body(ids_hbm, tok_tab, out_hbm, ids_v, *scratch):
    wid = lax.axis_index("s") * NC + lax.axis_index("c")
    b = wid // WPB
    k = wid % WPB
    base = b * HSEQ + k * TOK_PER_W
    bufs = scratch[:NBUF]
    gsems = scratch[NBUF:2 * NBUF]
    osems = scratch[2 * NBUF:3 * NBUF]

    pltpu.sync_copy(ids_hbm.at[pl.ds(base, TOK_PER_W)], ids_v)

    def issue_gather(n, p):
        idx = ids_v.at[pl.ds(n * CH, CH)]
        pltpu.async_copy(tok_tab.at[idx], bufs[p], gsems[p])

    def wait_gather(p):
        pltpu.make_async_copy(tok_tab.at[pl.ds(0, CH)], bufs[p],
                              gsems[p]).wait()

    def issue_store(n, p):
        off = pl.multiple_of(base + n * CH, CH)
        pltpu.async_copy(bufs[p], out_hbm.at[pl.ds(off, CH)], osems[p])

    def wait_store(p):
        pltpu.make_async_copy(bufs[p], out_hbm.at[pl.ds(0, CH)],
                              osems[p]).wait()

    L = min(LOOKAHEAD, NCH)
    for n in range(L):
        issue_gather(n, n % NBUF)
    for n in range(NCH):
        wait_gather(n % NBUF)
        issue_store(n, n % NBUF)
        m = n + L
        if m < NCH:
            q = m % NBUF
            if m >= NBUF:
                wait_store(q)
            issue_gather(m, q)
    for k in range(max(0, NCH - NBUF), NCH):
        wait_store(k % NBUF)


def _sc_gather(ids_s, token_table):
    mesh = plsc.VectorSubcoreMesh(
        core_axis_name="c", subcore_axis_name="s",
        num_cores=NC, num_subcores=NS)
    run = pl.kernel(
        _sc_body,
        out_type=jax.ShapeDtypeStruct((NTOKK, DMODEL), jnp.float32),
        mesh=mesh,
        compiler_params=pltpu.CompilerParams(needs_layout_passes=False),
        scratch_types=(
            [pltpu.VMEM((TOK_PER_W,), jnp.int32)]
            + [pltpu.VMEM((CH, DMODEL), jnp.float32)] * NBUF
            + [pltpu.SemaphoreType.DMA] * (2 * NBUF)
        ),
    )
    return run(ids_s, token_table)


def _tc_body(x_ref, pos_ref, seg_ref, segtab_ref, lnw_ref,
             lnb_ref, out_ref):
    seg = seg_ref[...]
    lanes = lax.broadcasted_iota(jnp.int32, (BS, 3), 1)
    onehot = (seg == lanes).astype(jnp.float32)
    seg_emb = jnp.dot(onehot, segtab_ref[...],
                      preferred_element_type=jnp.float32)
    x = (x_ref[...] + pos_ref[...]) + seg_emb
    ones = jnp.ones((DMODEL, 1), jnp.float32)
    s1 = jnp.dot(x, ones, preferred_element_type=jnp.float32)
    s2 = jnp.dot(x * x, ones, preferred_element_type=jnp.float32)
    mean = s1 * (1.0 / DMODEL)
    var = s2 * (1.0 / DMODEL) - mean * mean
    rstd = lax.rsqrt(var + EPS)
    w = lnw_ref[...]
    out_ref[...] = (x * rstd - mean * rstd) * w + lnb_ref[...]


def _tc_finish(prev, x, segs_s, position_table, segment_table, lnw, lnb, s):
    hblk = HSEQ // BS
    gridk = BATCH * hblk

    def xmap(i):
        return ((i % BATCH) * hblk + i // BATCH, 0)

    def posmap(i):
        return (s * hblk + i // BATCH, 0)

    def omap(i):
        return ((i % BATCH) * SBLK + s * hblk + i // BATCH, 0)

    def body(*refs):
        if prev is None:
            _tc_body(*refs)
        else:
            _tc_body(*refs[1:])

    in_specs = [
        pl.BlockSpec((BS, DMODEL), xmap),
        pl.BlockSpec((BS, DMODEL), posmap),
        pl.BlockSpec((BS, 1), omap),
        pl.BlockSpec((3, DMODEL), lambda i: (0, 0)),
        pl.BlockSpec((1, DMODEL), lambda i: (0, 0)),
        pl.BlockSpec((1, DMODEL), lambda i: (0, 0)),
    ]
    args = [x, position_table, segs_s, segment_table,
            lnw.reshape(1, DMODEL), lnb.reshape(1, DMODEL)]
    aliases = {}
    if prev is not None:
        in_specs = [pl.BlockSpec(memory_space=pl.ANY)] + in_specs
        args = [prev] + args
        aliases = {0: 0}
    return pl.pallas_call(
        body,
        grid=(gridk,),
        in_specs=in_specs,
        out_specs=pl.BlockSpec((BS, DMODEL), omap),
        out_shape=jax.ShapeDtypeStruct((NTOK, DMODEL), jnp.float32),
        input_output_aliases=aliases,
        compiler_params=pltpu.CompilerParams(
            dimension_semantics=("arbitrary",)),
    )(*args)


@jax.jit
def kernel(input_ids, segments, token_table, position_table, segment_table,
           ln_weight, ln_bias):
    segs = segments.reshape(NTOK, 1).astype(jnp.int32)
    ids2 = input_ids.reshape(BATCH, SEQLEN).astype(jnp.int32)
    gathered = [
        _sc_gather(ids2[:, s * HSEQ:(s + 1) * HSEQ].reshape(NTOKK),
                   token_table)
        for s in range(NSLICE)
    ]
    out = None
    for s in range(NSLICE):
        out = _tc_finish(out, gathered[s], segs,
                         position_table, segment_table,
                         ln_weight, ln_bias, s)
    return out.reshape(BATCH, SEQLEN, DMODEL)


# speedup vs baseline: 1.0837x; 1.0837x over previous
import functools

import jax
import jax.numpy as jnp
from jax import lax
from jax.experimental import pallas as pl
from jax.experimental.pallas import tpu as pltpu
from jax.experimental.pallas import tpu_sc as plsc

NTOKEN = 100000
DMODEL = 1024
SEQLEN = 2048
BATCH = 4
EPS = 1e-5

NC = 2
NS = 16
NW = NC * NS
NTOK = BATCH * SEQLEN

SC_TOK = NTOK

NSLICE = 2
NTOKK = SC_TOK // NSLICE
HSEQ = SEQLEN // NSLICE
WPB = NW // BATCH
TOK_PER_W = HSEQ // WPB
CH = 16
NCH = TOK_PER_W // CH
NBUF = 6
LOOKAHEAD = 4

BS = 1024
SBLK = SEQLEN // BS
GRIDK = NTOKK // BS


def _sc_body(ids_hbm, tok_tab, out_hbm, ids_v, *scratch):
    wid = lax.axis_index("s") * NC + lax.axis_index("c")
    b = wid // WPB
    k = wid % WPB
    base = b * HSEQ + k * TOK_PER_W
    bufs = scratch[:NBUF]
    gsems = scratch[NBUF:2 * NBUF]
    osems = scratch[2 * NBUF:3 * NBUF]

    pltpu.sync_copy(ids_hbm.at[pl.ds(base, TOK_PER_W)], ids_v)

    def issue_gather(n, p):
        idx = ids_v.at[pl.ds(n * CH, CH)]
        pltpu.async_copy(tok_tab.at[idx], bufs[p], gsems[p])

    def wait_gather(p):
        pltpu.make_async_copy(tok_tab.at[pl.ds(0, CH)], bufs[p],
                              gsems[p]).wait()

    def issue_store(n, p):
        off = pl.multiple_of(base + n * CH, CH)
        pltpu.async_copy(bufs[p], out_hbm.at[pl.ds(off, CH)], osems[p])

    def wait_store(p):
        pltpu.make_async_copy(bufs[p], out_hbm.at[pl.ds(0, CH)],
                              osems[p]).wait()

    L = min(LOOKAHEAD, NCH)
    for n in range(L):
        issue_gather(n, n % NBUF)
    for n in range(NCH):
        wait_gather(n % NBUF)
        issue_store(n, n % NBUF)
        m = n + L
        if m < NCH:
            q = m % NBUF
            if m >= NBUF:
                wait_store(q)
            issue_gather(m, q)
    for k in range(max(0, NCH - NBUF), NCH):
        wait_store(k % NBUF)


def _sc_gather(ids_s, token_table):
    mesh = plsc.VectorSubcoreMesh(
        core_axis_name="c", subcore_axis_name="s",
        num_cores=NC, num_subcores=NS)
    run = pl.kernel(
        _sc_body,
        out_type=jax.ShapeDtypeStruct((NTOKK, DMODEL), jnp.float32),
        mesh=mesh,
        compiler_params=pltpu.CompilerParams(needs_layout_passes=False),
        scratch_types=(
            [pltpu.VMEM((TOK_PER_W,), jnp.int32)]
            + [pltpu.VMEM((CH, DMODEL), jnp.float32)] * NBUF
            + [pltpu.SemaphoreType.DMA] * (2 * NBUF)
        ),
    )
    return run(ids_s, token_table)


def _tc_body(x_ref, pos_ref, seg_ref, segtab_ref, lnw_ref,
             lnb_ref, out_ref):
    seg = seg_ref[...]
    lanes = lax.broadcasted_iota(jnp.int32, (BS, 3), 1)
    onehot = (seg == lanes).astype(jnp.float32)
    seg_emb = jnp.dot(onehot, segtab_ref[...],
                      preferred_element_type=jnp.float32)
    x = (x_ref[...] + pos_ref[...]) + seg_emb
    ones = jnp.ones((DMODEL, 1), jnp.float32)
    s1 = jnp.dot(x, ones, preferred_element_type=jnp.float32)
    s2 = jnp.dot(x * x, ones, preferred_element_type=jnp.float32)
    mean = s1 * (1.0 / DMODEL)
    var = s2 * (1.0 / DMODEL) - mean * mean
    rstd = lax.rsqrt(var + EPS)
    w = lnw_ref[...]
    out_ref[...] = (x * rstd - mean * rstd) * w + lnb_ref[...]


def _tc_finish(prev, x, segs_s, position_table, segment_table, lnw, lnb, s):
    hblk = HSEQ // BS
    gridk = BATCH * hblk

    def xmap(i):
        return ((i % BATCH) * hblk + i // BATCH, 0)

    def posmap(i):
        return (s * hblk + i // BATCH, 0)

    def omap(i):
        return ((i % BATCH) * SBLK + s * hblk + i // BATCH, 0)

    def body(*refs):
        if prev is None:
            _tc_body(*refs)
        else:
            _tc_body(*refs[1:])

    in_specs = [
        pl.BlockSpec((BS, DMODEL), xmap),
        pl.BlockSpec((BS, DMODEL), posmap),
        pl.BlockSpec((BS, 1), omap),
        pl.BlockSpec((3, DMODEL), lambda i: (0, 0)),
        pl.BlockSpec((1, DMODEL), lambda i: (0, 0)),
        pl.BlockSpec((1, DMODEL), lambda i: (0, 0)),
    ]
    args = [x, position_table, segs_s, segment_table,
            lnw.reshape(1, DMODEL), lnb.reshape(1, DMODEL)]
    aliases = {}
    if prev is not None:
        in_specs = [pl.BlockSpec(memory_space=pl.ANY)] + in_specs
        args = [prev] + args
        aliases = {0: 0}
    return pl.pallas_call(
        body,
        grid=(gridk,),
        in_specs=in_specs,
        out_specs=pl.BlockSpec((BS, DMODEL), omap),
        out_shape=jax.ShapeDtypeStruct((NTOK, DMODEL), jnp.float32),
        input_output_aliases=aliases,
        compiler_params=pltpu.CompilerParams(
            dimension_semantics=("arbitrary",)),
    )(*args)


@jax.jit
def kernel(input_ids, segments, token_table, position_table, segment_table,
           ln_weight, ln_bias):
    segs = segments.reshape(NTOK, 1).astype(jnp.int32)
    ids2 = input_ids.reshape(BATCH, SEQLEN).astype(jnp.int32)
    gathered = [
        _sc_gather(ids2[:, s * HSEQ:(s + 1) * HSEQ].reshape(NTOKK),
                   token_table)
        for s in range(NSLICE)
    ]
    out = None
    for s in range(NSLICE):
        out = _tc_finish(out, gathered[s], segs,
                         position_table, segment_table,
                         ln_weight, ln_bias, s)
    return out.reshape(BATCH, SEQLEN, DMODEL)
